# baseline (device time: 43068 ns/iter reference)
import jax
import jax.numpy as jnp
from jax import lax
from jax.experimental import pallas as pl
from jax.experimental.pallas import tpu as pltpu

N_DEV = 16
SQ = 512
D = 1024
DH = 128
HQ_LOCAL = 8
SCALE = 0.08838834764831843

CW = 512
MASKS = ((1, 4, 8, 3), (4, 3, 1, 8))
LVLROWS = (256, 128, 128, 128)
LVLOFF = (0, 256, 384, 512)
BF16 = jnp.bfloat16


def kernel(x, Wq, Wo, Wk, Wv):
    idx = lax.axis_index("i")
    x2 = x.reshape(SQ, D)
    wk_l = lax.dynamic_slice(Wk, (0, idx * (2 * DH)), (D, 2 * DH))
    wv_l = lax.dynamic_slice(Wv, (0, idx * (2 * DH)), (D, 2 * DH))

    def body(x_ref, wq_ref, wo_ref, wk_ref, wv_ref, out_ref,
             acc_ref, rs_buf, outb_ref,
             rs_send, rs_recv, ag_send, ag_recv):
        my = lax.axis_index("i")
        b0 = jnp.bitwise_and(my, 1)
        b1 = jnp.bitwise_and(lax.shift_right_logical(my, 1), 1)
        b2 = jnp.bitwise_and(lax.shift_right_logical(my, 2), 1)
        b3 = jnp.bitwise_and(lax.shift_right_logical(my, 3), 1)
        BITS = ((jnp.bitwise_xor(b0, b1), b2, b3, b1),
                (b2, b1, b0, b3))

        bar = pltpu.get_barrier_semaphore()
        for mask in (1, 3, 4, 8):
            pl.semaphore_signal(bar, inc=1,
                                device_id=(jnp.bitwise_xor(my, mask),),
                                device_id_type=pl.DeviceIdType.MESH)
        pl.semaphore_wait(bar, 4)

        f32 = jnp.float32
        xv = x_ref[:].astype(BF16)
        q = jnp.dot(xv, wq_ref[:].astype(BF16),
                    preferred_element_type=f32).astype(BF16)
        k = jnp.dot(xv, wk_ref[:].astype(BF16),
                    preferred_element_type=f32).astype(BF16)
        v = jnp.dot(xv, wv_ref[:].astype(BF16),
                    preferred_element_type=f32).astype(BF16)
        outs = []
        for h in range(HQ_LOCAL):
            g = h // 4
            qh = q[:, h * DH:(h + 1) * DH]
            kg = k[:, g * DH:(g + 1) * DH]
            vg = v[:, g * DH:(g + 1) * DH]
            s = lax.dot_general(qh, kg, (((1,), (1,)), ((), ())),
                                preferred_element_type=f32) * SCALE
            m = jnp.max(s, axis=-1, keepdims=True)
            p = jnp.exp(s - m)
            l = jnp.sum(p, axis=-1, keepdims=True)
            o = jnp.dot(p.astype(BF16), vg, preferred_element_type=f32)
            outs.append((o / l).astype(BF16))
        a = jnp.concatenate(outs, axis=1)
        acc_ref[:] = jnp.dot(a, wo_ref[:].astype(BF16),
                             preferred_element_type=f32).astype(BF16)

        seg = [jnp.int32(0), jnp.int32(0)]
        for t in range(4):
            rows, ro = LVLROWS[t], LVLOFF[t]
            butterfly = t >= 2
            started = []
            for s in range(2):
                partner = jnp.bitwise_xor(my, MASKS[s][t])
                if butterfly:
                    send_off = seg[s]
                    keep_off = seg[s]
                else:
                    bit = BITS[s][t]
                    send_off = seg[s] + (1 - bit) * rows
                    keep_off = seg[s] + bit * rows
                rdma = pltpu.make_async_remote_copy(
                    src_ref=acc_ref.at[pl.ds(send_off, rows),
                                       pl.ds(s * CW, CW)],
                    dst_ref=rs_buf.at[s, pl.ds(ro, rows)],
                    send_sem=rs_send.at[s, t],
                    recv_sem=rs_recv.at[s, t],
                    device_id=(partner,),
                    device_id_type=pl.DeviceIdType.MESH,
                )
                rdma.start()
                started.append((rdma, s, keep_off))
                seg[s] = keep_off
            for rdma, s, keep_off in started:
                rdma.wait_recv()
                if butterfly:
                    rdma.wait_send()
                acc_ref[pl.ds(keep_off, rows), pl.ds(s * CW, CW)] = (
                    acc_ref[pl.ds(keep_off, rows), pl.ds(s * CW, CW)]
                    + rs_buf[s, pl.ds(ro, rows), :])
            if not butterfly:
                for rdma, _, _ in started:
                    rdma.wait_send()

        for s in range(2):
            red = acc_ref[pl.ds(seg[s], 128), pl.ds(s * CW, CW)]
            outb_ref[pl.ds(seg[s], 128), pl.ds(s * CW, CW)] = red
            out_ref[pl.ds(seg[s], 128), pl.ds(s * CW, CW)] = (
                red.astype(jnp.float32))

        cur = [seg[0], seg[1]]
        for t in (1, 0):
            clen = LVLROWS[t]
            started = []
            for s in range(2):
                partner = jnp.bitwise_xor(my, MASKS[s][t])
                bit = BITS[s][t]
                parent = cur[s] - bit * clen
                partner_off = parent + (1 - bit) * clen
                rdma = pltpu.make_async_remote_copy(
                    src_ref=outb_ref.at[pl.ds(cur[s], clen),
                                        pl.ds(s * CW, CW)],
                    dst_ref=outb_ref.at[pl.ds(cur[s], clen),
                                        pl.ds(s * CW, CW)],
                    send_sem=ag_send.at[s, t],
                    recv_sem=ag_recv.at[s, t],
                    device_id=(partner,),
                    device_id_type=pl.DeviceIdType.MESH,
                )
                rdma.start()
                started.append((rdma, s, partner_off))
                cur[s] = parent
            for rdma, s, partner_off in started:
                rdma.wait_recv()
                out_ref[pl.ds(partner_off, clen), pl.ds(s * CW, CW)] = (
                    outb_ref[pl.ds(partner_off, clen),
                             pl.ds(s * CW, CW)].astype(jnp.float32))
            for rdma, _, _ in started:
                rdma.wait_send()

    out = pl.pallas_call(
        body,
        out_shape=jax.ShapeDtypeStruct((SQ, D), jnp.float32),
        in_specs=[
            pl.BlockSpec(memory_space=pltpu.VMEM),
            pl.BlockSpec(memory_space=pltpu.VMEM),
            pl.BlockSpec(memory_space=pltpu.VMEM),
            pl.BlockSpec(memory_space=pltpu.VMEM),
            pl.BlockSpec(memory_space=pltpu.VMEM),
        ],
        out_specs=pl.BlockSpec(memory_space=pltpu.VMEM),
        scratch_shapes=[
            pltpu.VMEM((SQ, D), BF16),
            pltpu.VMEM((2, 640, CW), BF16),
            pltpu.VMEM((SQ, D), BF16),
            pltpu.SemaphoreType.DMA((2, 4)),
            pltpu.SemaphoreType.DMA((2, 4)),
            pltpu.SemaphoreType.DMA((2, 2)),
            pltpu.SemaphoreType.DMA((2, 2)),
        ],
        compiler_params=pltpu.CompilerParams(collective_id=0),
    )(x2, Wq, Wo, wk_l, wv_l)
    return out.reshape(1, SQ, D)


# device time: 41342 ns/iter; 1.0417x vs baseline; 1.0417x over previous
import jax
import jax.numpy as jnp
from jax import lax
from jax.experimental import pallas as pl
from jax.experimental.pallas import tpu as pltpu

N_DEV = 16
SQ = 512
D = 1024
DH = 128
HQ_LOCAL = 8
SCALE = 0.08838834764831843

CW = 512
MASKS = ((1, 4, 8, 3), (4, 3, 1, 8))
LVLROWS = (256, 128, 128, 128)
LVLOFF = (0, 256, 384, 512)
BF16 = jnp.bfloat16


def kernel(x, Wq, Wo, Wk, Wv):
    idx = lax.axis_index("i")
    x2 = x.reshape(SQ, D).astype(BF16)
    wk_l = lax.dynamic_slice(Wk, (0, idx * (2 * DH)), (D, 2 * DH)).astype(BF16)
    wv_l = lax.dynamic_slice(Wv, (0, idx * (2 * DH)), (D, 2 * DH)).astype(BF16)

    def body(x_ref, wq_ref, wk_ref, wv_ref, wo_ref, out_ref,
             acc_ref, rs_buf, outb_ref, rs_send, rs_recv, ag_send, ag_recv):
        my = lax.axis_index("i")
        b0 = jnp.bitwise_and(my, 1)
        b1 = jnp.bitwise_and(lax.shift_right_logical(my, 1), 1)
        b2 = jnp.bitwise_and(lax.shift_right_logical(my, 2), 1)
        b3 = jnp.bitwise_and(lax.shift_right_logical(my, 3), 1)
        BITS = ((jnp.bitwise_xor(b0, b1), b2, b3, b1),
                (b2, b1, b0, b3))

        bar = pltpu.get_barrier_semaphore()
        for mask in (1, 3, 4, 8):
            pl.semaphore_signal(bar, inc=1,
                                device_id=(jnp.bitwise_xor(my, mask),),
                                device_id_type=pl.DeviceIdType.MESH)
        pl.semaphore_wait(bar, 4)

        xv = x_ref[:]
        f32 = jnp.float32
        q = jnp.dot(xv, wq_ref[:], preferred_element_type=f32).astype(BF16)
        k = jnp.dot(xv, wk_ref[:], preferred_element_type=f32).astype(BF16)
        v = jnp.dot(xv, wv_ref[:], preferred_element_type=f32).astype(BF16)
        outs = []
        for h in range(HQ_LOCAL):
            g = h // 4
            qh = q[:, h * DH:(h + 1) * DH]
            kg = k[:, g * DH:(g + 1) * DH]
            vg = v[:, g * DH:(g + 1) * DH]
            s = lax.dot_general(qh, kg, (((1,), (1,)), ((), ())),
                                preferred_element_type=f32) * SCALE
            m = jnp.max(s, axis=-1, keepdims=True)
            p = jnp.exp(s - m)
            l = jnp.sum(p, axis=-1, keepdims=True)
            o = jnp.dot(p.astype(BF16), vg, preferred_element_type=f32)
            outs.append((o / l).astype(BF16))
        a = jnp.concatenate(outs, axis=1)
        acc_ref[:] = jnp.dot(a, wo_ref[:], preferred_element_type=f32).astype(BF16)

        seg = [jnp.int32(0), jnp.int32(0)]
        deferred = []

        def start_rs(s, t):
            rows, ro = LVLROWS[t], LVLOFF[t]
            partner = jnp.bitwise_xor(my, MASKS[s][t])
            if t >= 2:
                send_off = keep_off = seg[s]
            else:
                bit = BITS[s][t]
                send_off = seg[s] + (1 - bit) * rows
                keep_off = seg[s] + bit * rows
            rdma = pltpu.make_async_remote_copy(
                src_ref=acc_ref.at[pl.ds(send_off, rows), pl.ds(s * CW, CW)],
                dst_ref=rs_buf.at[s, pl.ds(ro, rows)],
                send_sem=rs_send.at[s, t],
                recv_sem=rs_recv.at[s, t],
                device_id=(partner,),
                device_id_type=pl.DeviceIdType.MESH,
            )
            rdma.start()
            seg[s] = keep_off
            return rdma, t, keep_off

        def finish_rs(s, pend_s):
            rdma, t, keep_off = pend_s
            rows, ro = LVLROWS[t], LVLOFF[t]
            rdma.wait_recv()
            if t >= 2:
                rdma.wait_send()
            else:
                deferred.append(rdma)
            acc_ref[pl.ds(keep_off, rows), pl.ds(s * CW, CW)] = (
                acc_ref[pl.ds(keep_off, rows), pl.ds(s * CW, CW)]
                + rs_buf[s, pl.ds(ro, rows), :])

        cur = [None, None]

        def start_ag(s, t):
            clen = LVLROWS[t]
            partner = jnp.bitwise_xor(my, MASKS[s][t])
            bit = BITS[s][t]
            parent = cur[s] - bit * clen
            partner_off = parent + (1 - bit) * clen
            rdma = pltpu.make_async_remote_copy(
                src_ref=outb_ref.at[pl.ds(cur[s], clen), pl.ds(s * CW, CW)],
                dst_ref=outb_ref.at[pl.ds(cur[s], clen), pl.ds(s * CW, CW)],
                send_sem=ag_send.at[s, t],
                recv_sem=ag_recv.at[s, t],
                device_id=(partner,),
                device_id_type=pl.DeviceIdType.MESH,
            )
            rdma.start()
            cur[s] = parent
            return rdma, partner_off, clen

        def finish_ag(s, pend_s):
            rdma, partner_off, clen = pend_s
            rdma.wait_recv()
            deferred.append(rdma)
            out_ref[pl.ds(partner_off, clen), pl.ds(s * CW, CW)] = (
                outb_ref[pl.ds(partner_off, clen),
                         pl.ds(s * CW, CW)].astype(jnp.float32))

        pend = [start_rs(0, 0), start_rs(1, 0)]
        for t in range(1, 4):
            for s in (0, 1):
                finish_rs(s, pend[s])
                pend[s] = start_rs(s, t)
        for s in (0, 1):
            finish_rs(s, pend[s])
            red = acc_ref[pl.ds(seg[s], 128), pl.ds(s * CW, CW)]
            outb_ref[pl.ds(seg[s], 128), pl.ds(s * CW, CW)] = red
            out_ref[pl.ds(seg[s], 128), pl.ds(s * CW, CW)] = (
                red.astype(jnp.float32))
            cur[s] = seg[s]
            pend[s] = start_ag(s, 1)
        for s in (0, 1):
            finish_ag(s, pend[s])
            pend[s] = start_ag(s, 0)
        for s in (0, 1):
            finish_ag(s, pend[s])
        for rdma in deferred:
            rdma.wait_send()

    out = pl.pallas_call(
        body,
        out_shape=jax.ShapeDtypeStruct((SQ, D), jnp.float32),
        in_specs=[pl.BlockSpec(memory_space=pltpu.VMEM)] * 5,
        out_specs=pl.BlockSpec(memory_space=pltpu.VMEM),
        scratch_shapes=[
            pltpu.VMEM((SQ, D), BF16),
            pltpu.VMEM((2, 640, CW), BF16),
            pltpu.VMEM((SQ, D), BF16),
            pltpu.SemaphoreType.DMA((2, 4)),
            pltpu.SemaphoreType.DMA((2, 4)),
            pltpu.SemaphoreType.DMA((2, 2)),
            pltpu.SemaphoreType.DMA((2, 2)),
        ],
        compiler_params=pltpu.CompilerParams(collective_id=0),
    )(x2, Wq.astype(BF16), wk_l, wv_l, Wo.astype(BF16))
    return out.reshape(1, SQ, D)


# device time: 40821 ns/iter; 1.0550x vs baseline; 1.0128x over previous
import jax
import jax.numpy as jnp
from jax import lax
from jax.experimental import pallas as pl
from jax.experimental.pallas import tpu as pltpu

N_DEV = 16
SQ = 512
D = 1024
DH = 128
HQ_LOCAL = 8
SCALE = 0.08838834764831843

CW = 512
MASKS = ((1, 4, 8, 3), (4, 3, 1, 8))
LVLROWS = (256, 128, 128, 128)
LVLOFF = (0, 256, 384, 512)
BF16 = jnp.bfloat16


def kernel(x, Wq, Wo, Wk, Wv):
    idx = lax.axis_index("i")
    x2 = x.reshape(SQ, D).astype(BF16)
    wk_l = lax.dynamic_slice(Wk, (0, idx * (2 * DH)), (D, 2 * DH)).astype(BF16)
    wv_l = lax.dynamic_slice(Wv, (0, idx * (2 * DH)), (D, 2 * DH)).astype(BF16)

    def body(x_ref, wq_ref, wk_ref, wv_ref, wo_ref, out_ref,
             acc_ref, rs_buf, outb_ref, a_ref,
             rs_send, rs_recv, ag_send, ag_recv):
        my = lax.axis_index("i")
        b0 = jnp.bitwise_and(my, 1)
        b1 = jnp.bitwise_and(lax.shift_right_logical(my, 1), 1)
        b2 = jnp.bitwise_and(lax.shift_right_logical(my, 2), 1)
        b3 = jnp.bitwise_and(lax.shift_right_logical(my, 3), 1)
        BITS = ((jnp.bitwise_xor(b0, b1), b2, b3, b1),
                (b2, b1, b0, b3))

        bar = pltpu.get_barrier_semaphore()
        for mask in (1, 3, 4, 8):
            pl.semaphore_signal(bar, inc=1,
                                device_id=(jnp.bitwise_xor(my, mask),),
                                device_id_type=pl.DeviceIdType.MESH)
        pl.semaphore_wait(bar, 4)

        xv = x_ref[:]
        f32 = jnp.float32
        q = jnp.dot(xv, wq_ref[:], preferred_element_type=f32).astype(BF16)
        k = jnp.dot(xv, wk_ref[:], preferred_element_type=f32).astype(BF16)
        v = jnp.dot(xv, wv_ref[:], preferred_element_type=f32).astype(BF16)
        outs = []
        for h in range(HQ_LOCAL):
            g = h // 4
            qh = q[:, h * DH:(h + 1) * DH]
            kg = k[:, g * DH:(g + 1) * DH]
            vg = v[:, g * DH:(g + 1) * DH]
            s = lax.dot_general(qh, kg, (((1,), (1,)), ((), ())),
                                preferred_element_type=f32) * SCALE
            m = jnp.max(s, axis=-1, keepdims=True)
            p = jnp.exp(s - m)
            l = jnp.sum(p, axis=-1, keepdims=True)
            o = jnp.dot(p.astype(BF16), vg, preferred_element_type=f32)
            outs.append((o / l).astype(BF16))
        a_ref[:] = jnp.concatenate(outs, axis=1)

        seg = [jnp.int32(0), jnp.int32(0)]
        deferred = []

        def start_rs(s, t):
            rows, ro = LVLROWS[t], LVLOFF[t]
            partner = jnp.bitwise_xor(my, MASKS[s][t])
            if t >= 2:
                send_off = keep_off = seg[s]
            else:
                bit = BITS[s][t]
                send_off = seg[s] + (1 - bit) * rows
                keep_off = seg[s] + bit * rows
            rdma = pltpu.make_async_remote_copy(
                src_ref=acc_ref.at[pl.ds(send_off, rows), pl.ds(s * CW, CW)],
                dst_ref=rs_buf.at[s, pl.ds(ro, rows)],
                send_sem=rs_send.at[s, t],
                recv_sem=rs_recv.at[s, t],
                device_id=(partner,),
                device_id_type=pl.DeviceIdType.MESH,
            )
            rdma.start()
            seg[s] = keep_off
            return rdma, t, keep_off

        def finish_rs(s, pend_s):
            rdma, t, keep_off = pend_s
            rows, ro = LVLROWS[t], LVLOFF[t]
            rdma.wait_recv()
            if t >= 2:
                rdma.wait_send()
            else:
                deferred.append(rdma)
            acc_ref[pl.ds(keep_off, rows), pl.ds(s * CW, CW)] = (
                acc_ref[pl.ds(keep_off, rows), pl.ds(s * CW, CW)]
                + rs_buf[s, pl.ds(ro, rows), :])

        cur = [None, None]

        def start_ag(s, t):
            clen = LVLROWS[t]
            partner = jnp.bitwise_xor(my, MASKS[s][t])
            bit = BITS[s][t]
            parent = cur[s] - bit * clen
            partner_off = parent + (1 - bit) * clen
            rdma = pltpu.make_async_remote_copy(
                src_ref=outb_ref.at[pl.ds(cur[s], clen), pl.ds(s * CW, CW)],
                dst_ref=outb_ref.at[pl.ds(cur[s], clen), pl.ds(s * CW, CW)],
                send_sem=ag_send.at[s, t],
                recv_sem=ag_recv.at[s, t],
                device_id=(partner,),
                device_id_type=pl.DeviceIdType.MESH,
            )
            rdma.start()
            cur[s] = parent
            return rdma, partner_off, clen

        def finish_ag(s, pend_s):
            rdma, partner_off, clen = pend_s
            rdma.wait_recv()
            deferred.append(rdma)
            out_ref[pl.ds(partner_off, clen), pl.ds(s * CW, CW)] = (
                outb_ref[pl.ds(partner_off, clen),
                         pl.ds(s * CW, CW)].astype(jnp.float32))

        wo = wo_ref[:]
        pend = [None, None]

        def proj_quad(row_off, s):
            blk = a_ref[pl.ds(row_off, 256), :]
            acc_ref[pl.ds(row_off, 256), pl.ds(s * CW, CW)] = jnp.dot(
                blk, wo[:, s * CW:(s + 1) * CW],
                preferred_element_type=f32).astype(BF16)

        for s in (0, 1):
            proj_quad((1 - BITS[s][0]) * 256, s)
            pend[s] = start_rs(s, 0)
        for s in (0, 1):
            proj_quad(BITS[s][0] * 256, s)
        for t in range(1, 4):
            for s in (0, 1):
                finish_rs(s, pend[s])
                pend[s] = start_rs(s, t)
        for s in (0, 1):
            finish_rs(s, pend[s])
            red = acc_ref[pl.ds(seg[s], 128), pl.ds(s * CW, CW)]
            outb_ref[pl.ds(seg[s], 128), pl.ds(s * CW, CW)] = red
            out_ref[pl.ds(seg[s], 128), pl.ds(s * CW, CW)] = (
                red.astype(jnp.float32))
            cur[s] = seg[s]
            pend[s] = start_ag(s, 1)
        for s in (0, 1):
            finish_ag(s, pend[s])
            pend[s] = start_ag(s, 0)
        for s in (0, 1):
            finish_ag(s, pend[s])
        for rdma in deferred:
            rdma.wait_send()

    out = pl.pallas_call(
        body,
        out_shape=jax.ShapeDtypeStruct((SQ, D), jnp.float32),
        in_specs=[pl.BlockSpec(memory_space=pltpu.VMEM)] * 5,
        out_specs=pl.BlockSpec(memory_space=pltpu.VMEM),
        scratch_shapes=[
            pltpu.VMEM((SQ, D), BF16),
            pltpu.VMEM((2, 640, CW), BF16),
            pltpu.VMEM((SQ, D), BF16),
            pltpu.VMEM((SQ, D), BF16),
            pltpu.SemaphoreType.DMA((2, 4)),
            pltpu.SemaphoreType.DMA((2, 4)),
            pltpu.SemaphoreType.DMA((2, 2)),
            pltpu.SemaphoreType.DMA((2, 2)),
        ],
        compiler_params=pltpu.CompilerParams(collective_id=0),
    )(x2, Wq.astype(BF16), wk_l, wv_l, Wo.astype(BF16))
    return out.reshape(1, SQ, D)
